# Initial kernel scaffold; baseline (speedup 1.0000x reference)
import jax
import jax.numpy as jnp
from jax.experimental import pallas as pl


def kernel(x, edge_index, batch, W1, b1, W2, b2, W_fc, b_fc):
    raise NotImplementedError("write your pallas kernel here")


# speedup vs baseline: 13.2387x; 13.2387x over previous
import functools

import jax
import jax.numpy as jnp
from jax import lax
from jax.experimental import pallas as pl
from jax.experimental.pallas import tpu as pltpu
from jax.experimental.pallas import tpu_sc as plsc

_N = 10000
_E = 320000
_D = 128
_G = 128
_C = 10

_NC = 2
_NS = 16
_NW = _NC * _NS
_EPT = _E // _NW
_CHUNK = 80
_NCHUNK = _EPT // _CHUNK
_STRIPE = 200
_NSTRIPE = _N // _STRIPE
_NSTRIPE_CEIL = -(-_NSTRIPE // _NS)

_BR = 2000


def _sc_mesh():
    return plsc.VectorSubcoreMesh(core_axis_name="c", subcore_axis_name="s")


def _sc_degree(dst):
    @functools.partial(
        pl.kernel,
        out_type=(jax.ShapeDtypeStruct((_N,), jnp.float32),
                  jax.ShapeDtypeStruct((_N,), jnp.float32)),
        mesh=_sc_mesh(),
        scratch_types=[
            pltpu.VMEM((_CHUNK,), jnp.int32),
            pltpu.VMEM((_CHUNK,), jnp.float32),
            pltpu.VMEM((2000,), jnp.float32),
            pltpu.VMEM_SHARED((_N,), jnp.float32),
        ],
    )
    def k(dst_hbm, degA_hbm, degB_hbm, dst_v, ones_v, stage_v, deg_sh):
        cid = lax.axis_index("c")
        sid = lax.axis_index("s")
        wid = cid * _NS + sid

        def zbody(i, carry):
            stage_v[pl.ds(i * 16, 16)] = jnp.zeros((16,), jnp.float32)
            return carry

        lax.fori_loop(0, 2000 // 16, zbody, 0)

        @pl.when(sid < 5)
        def _zero():
            pltpu.sync_copy(stage_v, deg_sh.at[pl.ds(sid * 2000, 2000)])

        for j in range(_CHUNK // 16):
            ones_v[pl.ds(j * 16, 16)] = jnp.ones((16,), jnp.float32)

        plsc.subcore_barrier()

        base = wid * _EPT

        def body(i, carry):
            off = pl.multiple_of(base + i * _CHUNK, 8)
            pltpu.sync_copy(dst_hbm.at[pl.ds(off, _CHUNK)], dst_v)
            pltpu.sync_copy(ones_v, deg_sh.at[dst_v], add=True)
            return carry

        lax.fori_loop(0, _NCHUNK, body, 0)
        plsc.subcore_barrier()

        @pl.when(sid < 5)
        def _out():
            pltpu.sync_copy(deg_sh.at[pl.ds(sid * 2000, 2000)], stage_v)

            @pl.when(cid == 0)
            def _():
                pltpu.sync_copy(stage_v, degA_hbm.at[pl.ds(sid * 2000, 2000)])

            @pl.when(cid == 1)
            def _():
                pltpu.sync_copy(stage_v, degB_hbm.at[pl.ds(sid * 2000, 2000)])

    return k(dst)


def _sc_scatter(y, src, dst):
    @functools.partial(
        pl.kernel,
        out_type=(jax.ShapeDtypeStruct((_N, _D), jnp.float32),
                  jax.ShapeDtypeStruct((_N, _D), jnp.float32)),
        mesh=_sc_mesh(),
        scratch_types=[
            pltpu.VMEM((_CHUNK,), jnp.int32),
            pltpu.VMEM((_CHUNK,), jnp.int32),
            pltpu.VMEM((_CHUNK, _D), jnp.float32),
            pltpu.VMEM((_STRIPE, _D), jnp.float32),
            pltpu.VMEM_SHARED((_N, _D), jnp.float32),
            pltpu.SemaphoreType.DMA,
        ],
    )
    def k(y_hbm, src_hbm, dst_hbm, accA_hbm, accB_hbm,
          src_v, dst_v, rows_v, stage_v, acc_sh, sem):
        cid = lax.axis_index("c")
        sid = lax.axis_index("s")
        wid = cid * _NS + sid

        def zbody(i, carry):
            def zinner(j, c2):
                stage_v[i, pl.ds(j * 16, 16)] = jnp.zeros((16,), jnp.float32)
                return c2
            return lax.fori_loop(0, _D // 16, zinner, carry)

        lax.fori_loop(0, _STRIPE, zbody, 0)

        for j in range(_NSTRIPE_CEIL):
            st = sid + _NS * j

            @pl.when(st < _NSTRIPE)
            def _():
                row0 = pl.multiple_of(st * _STRIPE, 8)
                pltpu.sync_copy(stage_v, acc_sh.at[pl.ds(row0, _STRIPE)])

        plsc.subcore_barrier()

        base = wid * _EPT

        def body(i, carry):
            off = pl.multiple_of(base + i * _CHUNK, 8)
            pltpu.sync_copy(src_hbm.at[pl.ds(off, _CHUNK)], src_v)
            pltpu.sync_copy(dst_hbm.at[pl.ds(off, _CHUNK)], dst_v)
            pltpu.async_copy(y_hbm.at[src_v], rows_v, sem).wait()
            pltpu.sync_copy(rows_v, acc_sh.at[dst_v], add=True)
            return carry

        lax.fori_loop(0, _NCHUNK, body, 0)
        plsc.subcore_barrier()

        for j in range(_NSTRIPE_CEIL):
            st = sid + _NS * j

            @pl.when(st < _NSTRIPE)
            def _():
                row0 = pl.multiple_of(st * _STRIPE, 8)
                pltpu.sync_copy(acc_sh.at[pl.ds(row0, _STRIPE)], stage_v)

                @pl.when(cid == 0)
                def _():
                    pltpu.sync_copy(stage_v, accA_hbm.at[pl.ds(row0, _STRIPE)])

                @pl.when(cid == 1)
                def _():
                    pltpu.sync_copy(stage_v, accB_hbm.at[pl.ds(row0, _STRIPE)])

    return k(y, src, dst)


def _tc_mm1(x, W1, degA, degB):
    def body(x_ref, w_ref, dA_ref, dB_ref, y_ref):
        dinv = lax.rsqrt(dA_ref[...] + dB_ref[...] + 1.0)
        y_ref[...] = jnp.dot(x_ref[...], w_ref[...],
                             preferred_element_type=jnp.float32) * dinv

    return pl.pallas_call(
        body,
        grid=(_N // _BR,),
        in_specs=[
            pl.BlockSpec((_BR, _D), lambda i: (i, 0)),
            pl.BlockSpec((_D, _D), lambda i: (0, 0)),
            pl.BlockSpec((_BR, 1), lambda i: (i, 0)),
            pl.BlockSpec((_BR, 1), lambda i: (i, 0)),
        ],
        out_specs=pl.BlockSpec((_BR, _D), lambda i: (i, 0)),
        out_shape=jax.ShapeDtypeStruct((_N, _D), jnp.float32),
    )(x, W1, degA, degB)


def _tc_mm2(accA, accB, y1, degA, degB, b1, W2):
    def body(aA_ref, aB_ref, y_ref, dA_ref, dB_ref, b_ref, w_ref, y2_ref):
        dinv = lax.rsqrt(dA_ref[...] + dB_ref[...] + 1.0)
        h = dinv * (aA_ref[...] + aB_ref[...] + y_ref[...]) + b_ref[...]
        h = jnp.maximum(h, 0.0)
        y2_ref[...] = jnp.dot(h, w_ref[...],
                              preferred_element_type=jnp.float32) * dinv

    return pl.pallas_call(
        body,
        grid=(_N // _BR,),
        in_specs=[
            pl.BlockSpec((_BR, _D), lambda i: (i, 0)),
            pl.BlockSpec((_BR, _D), lambda i: (i, 0)),
            pl.BlockSpec((_BR, _D), lambda i: (i, 0)),
            pl.BlockSpec((_BR, 1), lambda i: (i, 0)),
            pl.BlockSpec((_BR, 1), lambda i: (i, 0)),
            pl.BlockSpec((1, _D), lambda i: (0, 0)),
            pl.BlockSpec((_D, _D), lambda i: (0, 0)),
        ],
        out_specs=pl.BlockSpec((_BR, _D), lambda i: (i, 0)),
        out_shape=jax.ShapeDtypeStruct((_N, _D), jnp.float32),
    )(accA, accB, y1, degA, degB, b1, W2)


def _tc_final(accA, accB, y2, degA, degB, b2, batch2d, W_fc, b_fc):
    def body(aA_ref, aB_ref, y_ref, dA_ref, dB_ref, b_ref, bt_ref,
             wfc_ref, bfc_ref, out_ref, gacc, cnt):
        i = pl.program_id(0)
        dinv = lax.rsqrt(dA_ref[...] + dB_ref[...] + 1.0)
        h2 = dinv * (aA_ref[...] + aB_ref[...] + y_ref[...]) + b_ref[...]
        gids = lax.broadcasted_iota(jnp.int32, (_BR, _G), 1)
        oh = (bt_ref[...] == gids).astype(jnp.float32)
        ps = lax.dot_general(oh, h2, (((0,), (0,)), ((), ())),
                             preferred_element_type=jnp.float32)
        pc = lax.dot_general(oh, jnp.ones((_BR, 1), jnp.float32),
                             (((0,), (0,)), ((), ())),
                             preferred_element_type=jnp.float32)

        @pl.when(i == 0)
        def _():
            gacc[...] = ps
            cnt[...] = pc

        @pl.when(i > 0)
        def _():
            gacc[...] += ps
            cnt[...] += pc

        @pl.when(i == pl.num_programs(0) - 1)
        def _():
            g = gacc[...] / jnp.maximum(cnt[...], 1.0)
            out_ref[...] = jnp.dot(g, wfc_ref[...],
                                   preferred_element_type=jnp.float32) + bfc_ref[...]

    return pl.pallas_call(
        body,
        grid=(_N // _BR,),
        in_specs=[
            pl.BlockSpec((_BR, _D), lambda i: (i, 0)),
            pl.BlockSpec((_BR, _D), lambda i: (i, 0)),
            pl.BlockSpec((_BR, _D), lambda i: (i, 0)),
            pl.BlockSpec((_BR, 1), lambda i: (i, 0)),
            pl.BlockSpec((_BR, 1), lambda i: (i, 0)),
            pl.BlockSpec((1, _D), lambda i: (0, 0)),
            pl.BlockSpec((_BR, 1), lambda i: (i, 0)),
            pl.BlockSpec((_D, _C), lambda i: (0, 0)),
            pl.BlockSpec((1, _C), lambda i: (0, 0)),
        ],
        out_specs=pl.BlockSpec((_G, _C), lambda i: (0, 0)),
        out_shape=jax.ShapeDtypeStruct((_G, _C), jnp.float32),
        scratch_shapes=[
            pltpu.VMEM((_G, _D), jnp.float32),
            pltpu.VMEM((_G, 1), jnp.float32),
        ],
    )(accA, accB, y2, degA, degB, b2, batch2d, W_fc, b_fc)


def kernel(x, edge_index, batch, W1, b1, W2, b2, W_fc, b_fc):
    src = edge_index[0].astype(jnp.int32)
    dst = edge_index[1].astype(jnp.int32)

    degA, degB = _sc_degree(dst)
    degA = degA.reshape(_N, 1)
    degB = degB.reshape(_N, 1)

    y1 = _tc_mm1(x, W1, degA, degB)
    a1A, a1B = _sc_scatter(y1, src, dst)
    y2 = _tc_mm2(a1A, a1B, y1, degA, degB, b1.reshape(1, _D), W2)
    a2A, a2B = _sc_scatter(y2, src, dst)
    return _tc_final(a2A, a2B, y2, degA, degB, b2.reshape(1, _D),
                     batch.astype(jnp.int32).reshape(_N, 1),
                     W_fc, b_fc.reshape(1, _C))
